# baseline (speedup 1.0000x reference)
import functools

import jax
import jax.numpy as jnp
from jax import lax
from jax.experimental import pallas as pl
from jax.experimental.pallas import tpu as pltpu
from jax.experimental.pallas import tpu_sc as plsc

N = 10000
E = 160000
D_IN = 256
D_HID = 256
D_OUT2 = 64
N_GRAPHS = 128

NC = 2
NS = 16
CHUNK = 128
E_PAD = 163840
DUMMY = N
ACC_ROWS = 10240
ROWS_PER_TILE = ACC_ROWS // NS
LAST_TILE = N // ROWS_PER_TILE
LAST_ROWS = N - LAST_TILE * ROWS_PER_TILE
DEG_W = 8

TILE_M = 2000
GRID_M = N // TILE_M

_f32 = jnp.float32
_i32 = jnp.int32


def _deg_body(col2_hbm, ones_hbm, zeros_hbm, out_hbm,
              coli_v, ones_v, zv, acc, sem):
    c = lax.axis_index("c")
    s = lax.axis_index("s")
    pltpu.sync_copy(zeros_hbm, zv)
    for z in range(ROWS_PER_TILE // CHUNK):
        pltpu.sync_copy(zv, acc.at[pl.ds(s * ROWS_PER_TILE + z * CHUNK, CHUNK)])
    pltpu.sync_copy(ones_hbm, ones_v)

    n_chunks = E_PAD // (NC * NS * CHUNK)
    cb = (c * NS + s) * n_chunks
    pltpu.sync_copy(col2_hbm.at[pl.ds(cb, n_chunks)], coli_v)
    plsc.subcore_barrier()

    @pl.loop(0, n_chunks)
    def _(g):
        pltpu.sync_copy(ones_v, acc.at[coli_v.at[g]], add=True)

    plsc.subcore_barrier()
    pltpu.sync_copy(acc.at[pl.ds(s * ROWS_PER_TILE, ROWS_PER_TILE)],
                    out_hbm.at[c, pl.ds(s * ROWS_PER_TILE, ROWS_PER_TILE)])


_deg_pass = functools.partial(
    pl.kernel,
    out_type=jax.ShapeDtypeStruct((NC, ACC_ROWS, DEG_W), _f32),
    mesh=plsc.VectorSubcoreMesh(core_axis_name="c", subcore_axis_name="s"),
    compiler_params=pltpu.CompilerParams(use_tc_tiling_on_sc=False),
    scratch_types=[
        pltpu.VMEM((E_PAD // (NC * NS * CHUNK), CHUNK), _i32),
        pltpu.VMEM((CHUNK, DEG_W), _f32),
        pltpu.VMEM((CHUNK, DEG_W), _f32),
        pltpu.VMEM_SHARED((ACC_ROWS, DEG_W), _f32),
        pltpu.SemaphoreType.DMA,
    ],
)(_deg_body)


NBUF = 2
GCH = 40


def _edge_agg_body(feature_split, ha_hbm, hb_hbm, row2_hbm, col2_hbm,
                   zeros_hbm, out_hbm, rowi_v, coli_v, acc,
                   bufs, sems, ssems):
    c = lax.axis_index("c")
    s = lax.axis_index("s")
    for z in range(ROWS_PER_TILE // CHUNK):
        pltpu.sync_copy(zeros_hbm,
                        acc.at[pl.ds(s * ROWS_PER_TILE + z * CHUNK, CHUNK)])

    if feature_split:
        n_chunks = E_PAD // (NS * CHUNK)
        cb = s * n_chunks
    else:
        n_chunks = E_PAD // (NC * NS * CHUNK)
        cb = (c * NS + s) * n_chunks

    plsc.subcore_barrier()

    def edge_loop(h_hbm):
        @pl.loop(0, n_chunks // GCH)
        def _(j):
            gb = cb + j * GCH
            pltpu.sync_copy(row2_hbm.at[pl.ds(gb, GCH)], rowi_v)
            pltpu.sync_copy(col2_hbm.at[pl.ds(gb, GCH)], coli_v)
            for i in range(NBUF):
                pltpu.async_copy(h_hbm.at[rowi_v.at[i]], bufs[i], sems[i])

            @pl.loop(0, GCH, step=NBUF)
            def _(g):
                for i in range(NBUF):
                    pltpu.make_async_copy(h_hbm.at[rowi_v.at[g + i]],
                                          bufs[i], sems[i]).wait()
                    pltpu.async_copy(bufs[i], acc.at[coli_v.at[g + i]],
                                     ssems[i], add=True)
                for i in range(NBUF):
                    pltpu.make_async_copy(bufs[i], acc.at[coli_v.at[g + i]],
                                          ssems[i]).wait()

                    @pl.when(g + NBUF + i < GCH)
                    def _():
                        pltpu.async_copy(h_hbm.at[rowi_v.at[g + NBUF + i]],
                                         bufs[i], sems[i])

    if feature_split:
        @pl.when(c == 0)
        def _():
            edge_loop(ha_hbm)

        @pl.when(c == 1)
        def _():
            edge_loop(hb_hbm)
    else:
        edge_loop(ha_hbm)

    plsc.subcore_barrier()

    @pl.when(s < LAST_TILE)
    def _():
        pltpu.sync_copy(acc.at[pl.ds(s * ROWS_PER_TILE, ROWS_PER_TILE)],
                        out_hbm.at[c, pl.ds(s * ROWS_PER_TILE, ROWS_PER_TILE)])

    @pl.when(s == LAST_TILE)
    def _():
        pltpu.sync_copy(acc.at[pl.ds(LAST_TILE * ROWS_PER_TILE, LAST_ROWS)],
                        out_hbm.at[c, pl.ds(LAST_TILE * ROWS_PER_TILE, LAST_ROWS)])


def _make_edge_agg(feature_split, D, dtype=_f32):
    def body(*refs):
        args, rest = refs[:9], refs[9:]
        _edge_agg_body(feature_split, *args, list(rest[:NBUF]),
                       list(rest[NBUF:2 * NBUF]), list(rest[2 * NBUF:]))

    return functools.partial(
        pl.kernel,
        out_type=jax.ShapeDtypeStruct((NC, N, D), dtype),
        mesh=plsc.VectorSubcoreMesh(core_axis_name="c", subcore_axis_name="s"),
        compiler_params=pltpu.CompilerParams(use_tc_tiling_on_sc=False),
        scratch_types=(
            [
                pltpu.VMEM((GCH, CHUNK), _i32),
                pltpu.VMEM((GCH, CHUNK), _i32),
                pltpu.VMEM_SHARED((ACC_ROWS, D), dtype),
            ]
            + [pltpu.VMEM((CHUNK, D), dtype) for _ in range(NBUF)]
            + [pltpu.SemaphoreType.DMA for _ in range(2 * NBUF)]
        ),
    )(body)


_bf16 = jnp.bfloat16
_agg1_pass = _make_edge_agg(False, D_HID, _bf16)
_agg2_pass = _make_edge_agg(False, D_OUT2, _bf16)


def _dis_from(dega_ref, degb_ref):
    deg = dega_ref[...] + degb_ref[...]
    return jnp.where(deg > 0.0, lax.rsqrt(deg), 0.0)


def _mm1_body(x_ref, w_ref, v_ref, dega_ref, degb_ref,
              h_ref, xv_ref):
    dis = _dis_from(dega_ref, degb_ref)
    xt = x_ref[...]
    h = jnp.dot(xt, w_ref[...], preferred_element_type=_f32) * dis
    h_ref[...] = h.astype(_bf16)
    xv_ref[...] = jnp.dot(xt, v_ref[...], preferred_element_type=_f32)


_mm1 = pl.pallas_call(
    _mm1_body,
    grid=(GRID_M,),
    in_specs=[
        pl.BlockSpec((TILE_M, D_IN), lambda i: (i, 0)),
        pl.BlockSpec((D_IN, D_HID), lambda i: (0, 0)),
        pl.BlockSpec((D_IN, D_HID), lambda i: (0, 0)),
        pl.BlockSpec((TILE_M, 1), lambda i: (i, 0)),
        pl.BlockSpec((TILE_M, 1), lambda i: (i, 0)),
    ],
    out_specs=[
        pl.BlockSpec((TILE_M, D_HID), lambda i: (i, 0)),
        pl.BlockSpec((TILE_M, D_HID), lambda i: (i, 0)),
    ],
    out_shape=[
        jax.ShapeDtypeStruct((N, D_HID), _bf16),
        jax.ShapeDtypeStruct((N, D_HID), _f32),
    ],
)


def _mm2_body(a_ref, b_ref, xv_ref, dega_ref, degb_ref, b1_ref,
              w2_ref, v2_ref, h2_ref, xv2_ref):
    dis = _dis_from(dega_ref, degb_ref)
    agg = a_ref[...].astype(_f32) + b_ref[...].astype(_f32)
    out1 = jnp.maximum(agg * dis + xv_ref[...] + b1_ref[...], 0.0)
    h2_ref[...] = (jnp.dot(out1, w2_ref[...],
                           preferred_element_type=_f32) * dis).astype(_bf16)
    xv2_ref[...] = jnp.dot(out1, v2_ref[...], preferred_element_type=_f32)


_mm2 = pl.pallas_call(
    _mm2_body,
    grid=(GRID_M,),
    in_specs=[
        pl.BlockSpec((TILE_M, D_HID), lambda i: (i, 0)),
        pl.BlockSpec((TILE_M, D_HID), lambda i: (i, 0)),
        pl.BlockSpec((TILE_M, D_HID), lambda i: (i, 0)),
        pl.BlockSpec((TILE_M, 1), lambda i: (i, 0)),
        pl.BlockSpec((TILE_M, 1), lambda i: (i, 0)),
        pl.BlockSpec((1, D_HID), lambda i: (0, 0)),
        pl.BlockSpec((D_HID, D_OUT2), lambda i: (0, 0)),
        pl.BlockSpec((D_HID, D_OUT2), lambda i: (0, 0)),
    ],
    out_specs=[
        pl.BlockSpec((TILE_M, D_OUT2), lambda i: (i, 0)),
        pl.BlockSpec((TILE_M, D_OUT2), lambda i: (i, 0)),
    ],
    out_shape=[
        jax.ShapeDtypeStruct((N, D_OUT2), _bf16),
        jax.ShapeDtypeStruct((N, D_OUT2), _f32),
    ],
)


def _final_body(a_ref, b_ref, xv2_ref, dega_ref, degb_ref, b2_ref,
                batch_ref, fcw_ref, fcb_ref, out_ref, pool_acc, cnt_acc):
    i = pl.program_id(0)

    @pl.when(i == 0)
    def _():
        pool_acc[...] = jnp.zeros_like(pool_acc)
        cnt_acc[...] = jnp.zeros_like(cnt_acc)

    dis = _dis_from(dega_ref, degb_ref)
    agg = a_ref[...].astype(_f32) + b_ref[...].astype(_f32)
    out2 = jnp.maximum(agg * dis + xv2_ref[...] + b2_ref[...], 0.0)
    bid = batch_ref[0]
    gid = lax.broadcasted_iota(_i32, (N_GRAPHS, TILE_M), 0)
    oh = (gid == bid).astype(_f32)
    pool_acc[...] += jnp.dot(oh, out2, preferred_element_type=_f32)
    cnt_acc[...] += jnp.sum(oh, axis=1, keepdims=True)

    @pl.when(i == GRID_M - 1)
    def _():
        pooled = pool_acc[...] / jnp.maximum(cnt_acc[...], 1.0)
        out_ref[...] = (jnp.dot(pooled, fcw_ref[...],
                                preferred_element_type=_f32) + fcb_ref[...])


_final = pl.pallas_call(
    _final_body,
    grid=(GRID_M,),
    in_specs=[
        pl.BlockSpec((TILE_M, D_OUT2), lambda i: (i, 0)),
        pl.BlockSpec((TILE_M, D_OUT2), lambda i: (i, 0)),
        pl.BlockSpec((TILE_M, D_OUT2), lambda i: (i, 0)),
        pl.BlockSpec((TILE_M, 1), lambda i: (i, 0)),
        pl.BlockSpec((TILE_M, 1), lambda i: (i, 0)),
        pl.BlockSpec((1, D_OUT2), lambda i: (0, 0)),
        pl.BlockSpec((1, 1, TILE_M), lambda i: (i, 0, 0)),
        pl.BlockSpec((D_OUT2, N_GRAPHS), lambda i: (0, 0)),
        pl.BlockSpec((1, N_GRAPHS), lambda i: (0, 0)),
    ],
    out_specs=pl.BlockSpec((N_GRAPHS, N_GRAPHS), lambda i: (0, 0)),
    out_shape=jax.ShapeDtypeStruct((N_GRAPHS, N_GRAPHS), _f32),
    scratch_shapes=[
        pltpu.VMEM((N_GRAPHS, D_OUT2), _f32),
        pltpu.VMEM((N_GRAPHS, 1), _f32),
    ],
)


@jax.jit
def kernel(x, edge_index, batch, W1, V1, b1, W2, V2, b2, fc_w, fc_b):
    row = edge_index[0].astype(_i32)
    col = edge_index[1].astype(_i32)
    rowp = jnp.concatenate([row, jnp.zeros((E_PAD - E,), _i32)])
    colp = jnp.concatenate([col, jnp.full((E_PAD - E,), DUMMY, _i32)])
    rowp = rowp.reshape(E_PAD // CHUNK, CHUNK)
    colp = colp.reshape(E_PAD // CHUNK, CHUNK)

    ones8 = jnp.ones((CHUNK, DEG_W), _f32)
    zeros8 = jnp.zeros((CHUNK, DEG_W), _f32)
    zeros256b = jnp.zeros((CHUNK, D_HID), _bf16)
    zeros64 = jnp.zeros((CHUNK, D_OUT2), _bf16)

    deg2 = _deg_pass(colp, ones8, zeros8)
    dega = deg2[0, :N, 0:1]
    degb = deg2[1, :N, 0:1]

    h1, xv1 = _mm1(x, W1, V1, dega, degb)
    agg1 = _agg1_pass(h1, h1, rowp, colp, zeros256b)
    h2, xv2 = _mm2(agg1[0], agg1[1], xv1, dega, degb,
                   b1.reshape(1, -1), W2, V2)
    agg2 = _agg2_pass(h2, h2, rowp, colp, zeros64)

    batch3d = batch.astype(_i32).reshape(GRID_M, 1, TILE_M)
    fcw_pad = jnp.zeros((D_OUT2, N_GRAPHS), _f32).at[:, :fc_w.shape[1]].set(fc_w)
    fcb_pad = jnp.zeros((1, N_GRAPHS), _f32).at[0, :fc_b.shape[0]].set(fc_b)
    out_pad = _final(agg2[0], agg2[1], xv2, dega, degb, b2.reshape(1, -1),
                     batch3d, fcw_pad, fcb_pad)
    return out_pad[:, :fc_b.shape[0]]


# speedup vs baseline: 1.0286x; 1.0286x over previous
import functools

import jax
import jax.numpy as jnp
from jax import lax
from jax.experimental import pallas as pl
from jax.experimental.pallas import tpu as pltpu
from jax.experimental.pallas import tpu_sc as plsc

N = 10000
E = 160000
D_IN = 256
D_HID = 256
D_OUT2 = 64
N_GRAPHS = 128

NC = 2
NS = 16
CHUNK = 128
E_PAD = 163840
DUMMY = N
ACC_ROWS = 10240
ROWS_PER_TILE = ACC_ROWS // NS
LAST_TILE = N // ROWS_PER_TILE
LAST_ROWS = N - LAST_TILE * ROWS_PER_TILE
DEG_W = 8

TILE_M = 2000
GRID_M = N // TILE_M

_f32 = jnp.float32
_i32 = jnp.int32


def _deg_body(col2_hbm, ones_hbm, zeros_hbm, out_hbm,
              coli_v, ones_v, zv, acc, sem):
    c = lax.axis_index("c")
    s = lax.axis_index("s")
    pltpu.sync_copy(zeros_hbm, zv)
    for z in range(ROWS_PER_TILE // CHUNK):
        pltpu.sync_copy(zv, acc.at[pl.ds(s * ROWS_PER_TILE + z * CHUNK, CHUNK)])
    pltpu.sync_copy(ones_hbm, ones_v)

    n_chunks = E_PAD // (NC * NS * CHUNK)
    cb = (c * NS + s) * n_chunks
    pltpu.sync_copy(col2_hbm.at[pl.ds(cb, n_chunks)], coli_v)
    plsc.subcore_barrier()

    @pl.loop(0, n_chunks)
    def _(g):
        pltpu.sync_copy(ones_v, acc.at[coli_v.at[g]], add=True)

    plsc.subcore_barrier()
    pltpu.sync_copy(acc.at[pl.ds(s * ROWS_PER_TILE, ROWS_PER_TILE)],
                    out_hbm.at[c, pl.ds(s * ROWS_PER_TILE, ROWS_PER_TILE)])


_deg_pass = functools.partial(
    pl.kernel,
    out_type=jax.ShapeDtypeStruct((NC, ACC_ROWS, DEG_W), _f32),
    mesh=plsc.VectorSubcoreMesh(core_axis_name="c", subcore_axis_name="s"),
    compiler_params=pltpu.CompilerParams(use_tc_tiling_on_sc=False),
    scratch_types=[
        pltpu.VMEM((E_PAD // (NC * NS * CHUNK), CHUNK), _i32),
        pltpu.VMEM((CHUNK, DEG_W), _f32),
        pltpu.VMEM((CHUNK, DEG_W), _f32),
        pltpu.VMEM_SHARED((ACC_ROWS, DEG_W), _f32),
        pltpu.SemaphoreType.DMA,
    ],
)(_deg_body)


NBUF = 2
GCH = 40


def _edge_agg_body(feature_split, ha_hbm, hb_hbm, row2_hbm, col2_hbm,
                   zeros_hbm, out_hbm, rowi_v, coli_v, acc,
                   bufs, sems):
    c = lax.axis_index("c")
    s = lax.axis_index("s")
    for z in range(ROWS_PER_TILE // CHUNK):
        pltpu.sync_copy(zeros_hbm,
                        acc.at[pl.ds(s * ROWS_PER_TILE + z * CHUNK, CHUNK)])

    if feature_split:
        n_chunks = E_PAD // (NS * CHUNK)
        cb = s * n_chunks
    else:
        n_chunks = E_PAD // (NC * NS * CHUNK)
        cb = (c * NS + s) * n_chunks

    plsc.subcore_barrier()

    def edge_loop(h_hbm):
        @pl.loop(0, n_chunks // GCH)
        def _(j):
            gb = cb + j * GCH
            pltpu.sync_copy(row2_hbm.at[pl.ds(gb, GCH)], rowi_v)
            pltpu.sync_copy(col2_hbm.at[pl.ds(gb, GCH)], coli_v)
            for i in range(NBUF):
                pltpu.async_copy(h_hbm.at[rowi_v.at[i]], bufs[i], sems[i])

            @pl.loop(0, GCH, step=NBUF)
            def _(g):
                for i in range(NBUF):
                    pltpu.make_async_copy(h_hbm.at[rowi_v.at[g + i]],
                                          bufs[i], sems[i]).wait()
                    pltpu.sync_copy(bufs[i], acc.at[coli_v.at[g + i]],
                                    add=True)

                    @pl.when(g + NBUF + i < GCH)
                    def _():
                        pltpu.async_copy(h_hbm.at[rowi_v.at[g + NBUF + i]],
                                         bufs[i], sems[i])

    if feature_split:
        @pl.when(c == 0)
        def _():
            edge_loop(ha_hbm)

        @pl.when(c == 1)
        def _():
            edge_loop(hb_hbm)
    else:
        edge_loop(ha_hbm)

    plsc.subcore_barrier()

    @pl.when(s < LAST_TILE)
    def _():
        pltpu.sync_copy(acc.at[pl.ds(s * ROWS_PER_TILE, ROWS_PER_TILE)],
                        out_hbm.at[c, pl.ds(s * ROWS_PER_TILE, ROWS_PER_TILE)])

    @pl.when(s == LAST_TILE)
    def _():
        pltpu.sync_copy(acc.at[pl.ds(LAST_TILE * ROWS_PER_TILE, LAST_ROWS)],
                        out_hbm.at[c, pl.ds(LAST_TILE * ROWS_PER_TILE, LAST_ROWS)])


def _make_edge_agg(feature_split, D, dtype=_f32):
    def body(*refs):
        args, rest = refs[:9], refs[9:]
        _edge_agg_body(feature_split, *args, list(rest[:NBUF]),
                       list(rest[NBUF:]))

    return functools.partial(
        pl.kernel,
        out_type=jax.ShapeDtypeStruct((NC, N, D), dtype),
        mesh=plsc.VectorSubcoreMesh(core_axis_name="c", subcore_axis_name="s"),
        compiler_params=pltpu.CompilerParams(use_tc_tiling_on_sc=False),
        scratch_types=(
            [
                pltpu.VMEM((GCH, CHUNK), _i32),
                pltpu.VMEM((GCH, CHUNK), _i32),
                pltpu.VMEM_SHARED((ACC_ROWS, D), dtype),
            ]
            + [pltpu.VMEM((CHUNK, D), dtype) for _ in range(NBUF)]
            + [pltpu.SemaphoreType.DMA for _ in range(NBUF)]
        ),
    )(body)


_bf16 = jnp.bfloat16
_agg1_pass = _make_edge_agg(False, D_HID, _bf16)
_agg2_pass = _make_edge_agg(False, D_OUT2, _bf16)


def _dis_from(dega_ref, degb_ref):
    deg = dega_ref[...] + degb_ref[...]
    return jnp.where(deg > 0.0, lax.rsqrt(deg), 0.0)


def _mm1_body(x_ref, w_ref, v_ref, dega_ref, degb_ref,
              h_ref, xv_ref):
    dis = _dis_from(dega_ref, degb_ref)
    xt = x_ref[...]
    h = jnp.dot(xt, w_ref[...], preferred_element_type=_f32) * dis
    h_ref[...] = h.astype(_bf16)
    xv_ref[...] = jnp.dot(xt, v_ref[...], preferred_element_type=_f32)


_mm1 = pl.pallas_call(
    _mm1_body,
    grid=(GRID_M,),
    in_specs=[
        pl.BlockSpec((TILE_M, D_IN), lambda i: (i, 0)),
        pl.BlockSpec((D_IN, D_HID), lambda i: (0, 0)),
        pl.BlockSpec((D_IN, D_HID), lambda i: (0, 0)),
        pl.BlockSpec((TILE_M, 1), lambda i: (i, 0)),
        pl.BlockSpec((TILE_M, 1), lambda i: (i, 0)),
    ],
    out_specs=[
        pl.BlockSpec((TILE_M, D_HID), lambda i: (i, 0)),
        pl.BlockSpec((TILE_M, D_HID), lambda i: (i, 0)),
    ],
    out_shape=[
        jax.ShapeDtypeStruct((N, D_HID), _bf16),
        jax.ShapeDtypeStruct((N, D_HID), _f32),
    ],
)


def _mm2_body(a_ref, b_ref, xv_ref, dega_ref, degb_ref, b1_ref,
              w2_ref, v2_ref, h2_ref, xv2_ref):
    dis = _dis_from(dega_ref, degb_ref)
    agg = a_ref[...].astype(_f32) + b_ref[...].astype(_f32)
    out1 = jnp.maximum(agg * dis + xv_ref[...] + b1_ref[...], 0.0)
    h2_ref[...] = (jnp.dot(out1, w2_ref[...],
                           preferred_element_type=_f32) * dis).astype(_bf16)
    xv2_ref[...] = jnp.dot(out1, v2_ref[...], preferred_element_type=_f32)


_mm2 = pl.pallas_call(
    _mm2_body,
    grid=(GRID_M,),
    in_specs=[
        pl.BlockSpec((TILE_M, D_HID), lambda i: (i, 0)),
        pl.BlockSpec((TILE_M, D_HID), lambda i: (i, 0)),
        pl.BlockSpec((TILE_M, D_HID), lambda i: (i, 0)),
        pl.BlockSpec((TILE_M, 1), lambda i: (i, 0)),
        pl.BlockSpec((TILE_M, 1), lambda i: (i, 0)),
        pl.BlockSpec((1, D_HID), lambda i: (0, 0)),
        pl.BlockSpec((D_HID, D_OUT2), lambda i: (0, 0)),
        pl.BlockSpec((D_HID, D_OUT2), lambda i: (0, 0)),
    ],
    out_specs=[
        pl.BlockSpec((TILE_M, D_OUT2), lambda i: (i, 0)),
        pl.BlockSpec((TILE_M, D_OUT2), lambda i: (i, 0)),
    ],
    out_shape=[
        jax.ShapeDtypeStruct((N, D_OUT2), _bf16),
        jax.ShapeDtypeStruct((N, D_OUT2), _f32),
    ],
)


def _final_body(a_ref, b_ref, xv2_ref, dega_ref, degb_ref, b2_ref,
                batch_ref, fcw_ref, fcb_ref, out_ref, pool_acc, cnt_acc):
    i = pl.program_id(0)

    @pl.when(i == 0)
    def _():
        pool_acc[...] = jnp.zeros_like(pool_acc)
        cnt_acc[...] = jnp.zeros_like(cnt_acc)

    dis = _dis_from(dega_ref, degb_ref)
    agg = a_ref[...].astype(_f32) + b_ref[...].astype(_f32)
    out2 = jnp.maximum(agg * dis + xv2_ref[...] + b2_ref[...], 0.0)
    bid = batch_ref[0]
    gid = lax.broadcasted_iota(_i32, (N_GRAPHS, TILE_M), 0)
    oh = (gid == bid).astype(_f32)
    pool_acc[...] += jnp.dot(oh, out2, preferred_element_type=_f32)
    cnt_acc[...] += jnp.sum(oh, axis=1, keepdims=True)

    @pl.when(i == GRID_M - 1)
    def _():
        pooled = pool_acc[...] / jnp.maximum(cnt_acc[...], 1.0)
        out_ref[...] = (jnp.dot(pooled, fcw_ref[...],
                                preferred_element_type=_f32) + fcb_ref[...])


_final = pl.pallas_call(
    _final_body,
    grid=(GRID_M,),
    in_specs=[
        pl.BlockSpec((TILE_M, D_OUT2), lambda i: (i, 0)),
        pl.BlockSpec((TILE_M, D_OUT2), lambda i: (i, 0)),
        pl.BlockSpec((TILE_M, D_OUT2), lambda i: (i, 0)),
        pl.BlockSpec((TILE_M, 1), lambda i: (i, 0)),
        pl.BlockSpec((TILE_M, 1), lambda i: (i, 0)),
        pl.BlockSpec((1, D_OUT2), lambda i: (0, 0)),
        pl.BlockSpec((1, 1, TILE_M), lambda i: (i, 0, 0)),
        pl.BlockSpec((D_OUT2, N_GRAPHS), lambda i: (0, 0)),
        pl.BlockSpec((1, N_GRAPHS), lambda i: (0, 0)),
    ],
    out_specs=pl.BlockSpec((N_GRAPHS, N_GRAPHS), lambda i: (0, 0)),
    out_shape=jax.ShapeDtypeStruct((N_GRAPHS, N_GRAPHS), _f32),
    scratch_shapes=[
        pltpu.VMEM((N_GRAPHS, D_OUT2), _f32),
        pltpu.VMEM((N_GRAPHS, 1), _f32),
    ],
)


@jax.jit
def kernel(x, edge_index, batch, W1, V1, b1, W2, V2, b2, fc_w, fc_b):
    row = edge_index[0].astype(_i32)
    col = edge_index[1].astype(_i32)
    rowp = jnp.concatenate([row, jnp.zeros((E_PAD - E,), _i32)])
    colp = jnp.concatenate([col, jnp.full((E_PAD - E,), DUMMY, _i32)])
    rowp = rowp.reshape(E_PAD // CHUNK, CHUNK)
    colp = colp.reshape(E_PAD // CHUNK, CHUNK)

    ones8 = jnp.ones((CHUNK, DEG_W), _f32)
    zeros8 = jnp.zeros((CHUNK, DEG_W), _f32)
    zeros256b = jnp.zeros((CHUNK, D_HID), _bf16)
    zeros64 = jnp.zeros((CHUNK, D_OUT2), _bf16)

    deg2 = _deg_pass(colp, ones8, zeros8)
    dega = deg2[0, :N, 0:1]
    degb = deg2[1, :N, 0:1]

    h1, xv1 = _mm1(x, W1, V1, dega, degb)
    agg1 = _agg1_pass(h1, h1, rowp, colp, zeros256b)
    h2, xv2 = _mm2(agg1[0], agg1[1], xv1, dega, degb,
                   b1.reshape(1, -1), W2, V2)
    agg2 = _agg2_pass(h2, h2, rowp, colp, zeros64)

    batch3d = batch.astype(_i32).reshape(GRID_M, 1, TILE_M)
    fcw_pad = jnp.zeros((D_OUT2, N_GRAPHS), _f32).at[:, :fc_w.shape[1]].set(fc_w)
    fcb_pad = jnp.zeros((1, N_GRAPHS), _f32).at[0, :fc_b.shape[0]].set(fc_b)
    out_pad = _final(agg2[0], agg2[1], xv2, dega, degb, b2.reshape(1, -1),
                     batch3d, fcw_pad, fcb_pad)
    return out_pad[:, :fc_b.shape[0]]


# speedup vs baseline: 1.0391x; 1.0102x over previous
import functools

import jax
import jax.numpy as jnp
from jax import lax
from jax.experimental import pallas as pl
from jax.experimental.pallas import tpu as pltpu
from jax.experimental.pallas import tpu_sc as plsc

N = 10000
E = 160000
D_IN = 256
D_HID = 256
D_OUT2 = 64
N_GRAPHS = 128

NC = 2
NS = 16
CHUNK = 128
E_PAD = 163840
DUMMY = N
ACC_ROWS = 10240
ROWS_PER_TILE = ACC_ROWS // NS
LAST_TILE = N // ROWS_PER_TILE
LAST_ROWS = N - LAST_TILE * ROWS_PER_TILE
DEG_W = 8

TILE_M = 2000
GRID_M = N // TILE_M

_f32 = jnp.float32
_i32 = jnp.int32


def _deg_body(col2_hbm, ones_hbm, zeros_hbm, out_hbm,
              coli_v, ones_v, zv, acc, sem):
    c = lax.axis_index("c")
    s = lax.axis_index("s")
    pltpu.sync_copy(zeros_hbm, zv)
    for z in range(ROWS_PER_TILE // CHUNK):
        pltpu.sync_copy(zv, acc.at[pl.ds(s * ROWS_PER_TILE + z * CHUNK, CHUNK)])
    pltpu.sync_copy(ones_hbm, ones_v)

    n_chunks = E_PAD // (NC * NS * CHUNK)
    cb = (c * NS + s) * n_chunks
    pltpu.sync_copy(col2_hbm.at[pl.ds(cb, n_chunks)], coli_v)
    plsc.subcore_barrier()

    @pl.loop(0, n_chunks)
    def _(g):
        pltpu.sync_copy(ones_v, acc.at[coli_v.at[g]], add=True)

    plsc.subcore_barrier()
    pltpu.sync_copy(acc.at[pl.ds(s * ROWS_PER_TILE, ROWS_PER_TILE)],
                    out_hbm.at[c, pl.ds(s * ROWS_PER_TILE, ROWS_PER_TILE)])


_deg_pass = functools.partial(
    pl.kernel,
    out_type=jax.ShapeDtypeStruct((NC, ACC_ROWS, DEG_W), _f32),
    mesh=plsc.VectorSubcoreMesh(core_axis_name="c", subcore_axis_name="s"),
    compiler_params=pltpu.CompilerParams(use_tc_tiling_on_sc=False),
    scratch_types=[
        pltpu.VMEM((E_PAD // (NC * NS * CHUNK), CHUNK), _i32),
        pltpu.VMEM((CHUNK, DEG_W), _f32),
        pltpu.VMEM((CHUNK, DEG_W), _f32),
        pltpu.VMEM_SHARED((ACC_ROWS, DEG_W), _f32),
        pltpu.SemaphoreType.DMA,
    ],
)(_deg_body)


NBUF = 2
GCH = 40


def _edge_agg_body(feature_split, ha_hbm, hb_hbm, row2_hbm, col2_hbm,
                   zeros_hbm, out_hbm, rowi_v, coli_v, acc,
                   bufs, sems):
    c = lax.axis_index("c")
    s = lax.axis_index("s")
    for z in range(ROWS_PER_TILE // CHUNK):
        pltpu.sync_copy(zeros_hbm,
                        acc.at[pl.ds(s * ROWS_PER_TILE + z * CHUNK, CHUNK)])

    if feature_split:
        n_chunks = E_PAD // (NS * CHUNK)
        cb = s * n_chunks
    else:
        n_chunks = E_PAD // (NC * NS * CHUNK)
        cb = (c * NS + s) * n_chunks

    plsc.subcore_barrier()

    def edge_loop(h_hbm):
        @pl.loop(0, n_chunks // GCH)
        def _(j):
            gb = cb + j * GCH
            pltpu.sync_copy(row2_hbm.at[pl.ds(gb, GCH)], rowi_v)
            pltpu.sync_copy(col2_hbm.at[pl.ds(gb, GCH)], coli_v)
            for i in range(NBUF):
                pltpu.async_copy(h_hbm.at[rowi_v.at[i]], bufs[i], sems[i])

            @pl.loop(0, GCH, step=NBUF)
            def _(g):
                for i in range(NBUF):
                    pltpu.make_async_copy(h_hbm.at[rowi_v.at[g + i]],
                                          bufs[i], sems[i]).wait()
                    pltpu.sync_copy(bufs[i], acc.at[coli_v.at[g + i]],
                                    add=True)

                    @pl.when(g + NBUF + i < GCH)
                    def _():
                        pltpu.async_copy(h_hbm.at[rowi_v.at[g + NBUF + i]],
                                         bufs[i], sems[i])

    if feature_split:
        @pl.when(c == 0)
        def _():
            edge_loop(ha_hbm)

        @pl.when(c == 1)
        def _():
            edge_loop(hb_hbm)
    else:
        edge_loop(ha_hbm)

    plsc.subcore_barrier()

    @pl.when(s < LAST_TILE)
    def _():
        pltpu.sync_copy(acc.at[pl.ds(s * ROWS_PER_TILE, ROWS_PER_TILE)],
                        out_hbm.at[c, pl.ds(s * ROWS_PER_TILE, ROWS_PER_TILE)])

    @pl.when(s == LAST_TILE)
    def _():
        pltpu.sync_copy(acc.at[pl.ds(LAST_TILE * ROWS_PER_TILE, LAST_ROWS)],
                        out_hbm.at[c, pl.ds(LAST_TILE * ROWS_PER_TILE, LAST_ROWS)])


def _make_edge_agg(feature_split, D, dtype=_f32):
    def body(*refs):
        args, rest = refs[:9], refs[9:]
        _edge_agg_body(feature_split, *args, list(rest[:NBUF]),
                       list(rest[NBUF:]))

    return functools.partial(
        pl.kernel,
        out_type=jax.ShapeDtypeStruct((NC, N, D), dtype),
        mesh=plsc.VectorSubcoreMesh(core_axis_name="c", subcore_axis_name="s"),
        compiler_params=pltpu.CompilerParams(use_tc_tiling_on_sc=False),
        scratch_types=(
            [
                pltpu.VMEM((GCH, CHUNK), _i32),
                pltpu.VMEM((GCH, CHUNK), _i32),
                pltpu.VMEM_SHARED((ACC_ROWS, D), dtype),
            ]
            + [pltpu.VMEM((CHUNK, D), dtype) for _ in range(NBUF)]
            + [pltpu.SemaphoreType.DMA for _ in range(NBUF)]
        ),
    )(body)


_bf16 = jnp.bfloat16
_agg1_pass = _make_edge_agg(False, D_HID, _bf16)
_agg2_pass = _make_edge_agg(False, D_OUT2, _bf16)


def _dis_from(dega_ref, degb_ref):
    deg = dega_ref[...] + degb_ref[...]
    return jnp.where(deg > 0.0, lax.rsqrt(deg), 0.0)


def _mmv_body(x_ref, v_ref, xv_ref):
    xv_ref[...] = jnp.dot(x_ref[...], v_ref[...], preferred_element_type=_f32)


_mmv = pl.pallas_call(
    _mmv_body,
    grid=(GRID_M,),
    in_specs=[
        pl.BlockSpec((TILE_M, D_IN), lambda i: (i, 0)),
        pl.BlockSpec((D_IN, D_HID), lambda i: (0, 0)),
    ],
    out_specs=pl.BlockSpec((TILE_M, D_HID), lambda i: (i, 0)),
    out_shape=jax.ShapeDtypeStruct((N, D_HID), _f32),
)


def _mm1_body(x_ref, w_ref, dega_ref, degb_ref, h_ref):
    dis = _dis_from(dega_ref, degb_ref)
    h = jnp.dot(x_ref[...], w_ref[...], preferred_element_type=_f32) * dis
    h_ref[...] = h.astype(_bf16)


_mm1 = pl.pallas_call(
    _mm1_body,
    grid=(GRID_M,),
    in_specs=[
        pl.BlockSpec((TILE_M, D_IN), lambda i: (i, 0)),
        pl.BlockSpec((D_IN, D_HID), lambda i: (0, 0)),
        pl.BlockSpec((TILE_M, 1), lambda i: (i, 0)),
        pl.BlockSpec((TILE_M, 1), lambda i: (i, 0)),
    ],
    out_specs=pl.BlockSpec((TILE_M, D_HID), lambda i: (i, 0)),
    out_shape=jax.ShapeDtypeStruct((N, D_HID), _bf16),
)


def _mm2_body(a_ref, b_ref, xv_ref, dega_ref, degb_ref, b1_ref,
              w2_ref, v2_ref, h2_ref, xv2_ref):
    dis = _dis_from(dega_ref, degb_ref)
    agg = a_ref[...].astype(_f32) + b_ref[...].astype(_f32)
    out1 = jnp.maximum(agg * dis + xv_ref[...] + b1_ref[...], 0.0)
    h2_ref[...] = (jnp.dot(out1, w2_ref[...],
                           preferred_element_type=_f32) * dis).astype(_bf16)
    xv2_ref[...] = jnp.dot(out1, v2_ref[...], preferred_element_type=_f32)


_mm2 = pl.pallas_call(
    _mm2_body,
    grid=(GRID_M,),
    in_specs=[
        pl.BlockSpec((TILE_M, D_HID), lambda i: (i, 0)),
        pl.BlockSpec((TILE_M, D_HID), lambda i: (i, 0)),
        pl.BlockSpec((TILE_M, D_HID), lambda i: (i, 0)),
        pl.BlockSpec((TILE_M, 1), lambda i: (i, 0)),
        pl.BlockSpec((TILE_M, 1), lambda i: (i, 0)),
        pl.BlockSpec((1, D_HID), lambda i: (0, 0)),
        pl.BlockSpec((D_HID, D_OUT2), lambda i: (0, 0)),
        pl.BlockSpec((D_HID, D_OUT2), lambda i: (0, 0)),
    ],
    out_specs=[
        pl.BlockSpec((TILE_M, D_OUT2), lambda i: (i, 0)),
        pl.BlockSpec((TILE_M, D_OUT2), lambda i: (i, 0)),
    ],
    out_shape=[
        jax.ShapeDtypeStruct((N, D_OUT2), _bf16),
        jax.ShapeDtypeStruct((N, D_OUT2), _f32),
    ],
)


def _final_body(a_ref, b_ref, xv2_ref, dega_ref, degb_ref, b2_ref,
                batch_ref, fcw_ref, fcb_ref, out_ref, pool_acc, cnt_acc):
    i = pl.program_id(0)

    @pl.when(i == 0)
    def _():
        pool_acc[...] = jnp.zeros_like(pool_acc)
        cnt_acc[...] = jnp.zeros_like(cnt_acc)

    dis = _dis_from(dega_ref, degb_ref)
    agg = a_ref[...].astype(_f32) + b_ref[...].astype(_f32)
    out2 = jnp.maximum(agg * dis + xv2_ref[...] + b2_ref[...], 0.0)
    bid = batch_ref[0]
    gid = lax.broadcasted_iota(_i32, (N_GRAPHS, TILE_M), 0)
    oh = (gid == bid).astype(_f32)
    pool_acc[...] += jnp.dot(oh, out2, preferred_element_type=_f32)
    cnt_acc[...] += jnp.sum(oh, axis=1, keepdims=True)

    @pl.when(i == GRID_M - 1)
    def _():
        pooled = pool_acc[...] / jnp.maximum(cnt_acc[...], 1.0)
        out_ref[...] = (jnp.dot(pooled, fcw_ref[...],
                                preferred_element_type=_f32) + fcb_ref[...])


_final = pl.pallas_call(
    _final_body,
    grid=(GRID_M,),
    in_specs=[
        pl.BlockSpec((TILE_M, D_OUT2), lambda i: (i, 0)),
        pl.BlockSpec((TILE_M, D_OUT2), lambda i: (i, 0)),
        pl.BlockSpec((TILE_M, D_OUT2), lambda i: (i, 0)),
        pl.BlockSpec((TILE_M, 1), lambda i: (i, 0)),
        pl.BlockSpec((TILE_M, 1), lambda i: (i, 0)),
        pl.BlockSpec((1, D_OUT2), lambda i: (0, 0)),
        pl.BlockSpec((1, 1, TILE_M), lambda i: (i, 0, 0)),
        pl.BlockSpec((D_OUT2, N_GRAPHS), lambda i: (0, 0)),
        pl.BlockSpec((1, N_GRAPHS), lambda i: (0, 0)),
    ],
    out_specs=pl.BlockSpec((N_GRAPHS, N_GRAPHS), lambda i: (0, 0)),
    out_shape=jax.ShapeDtypeStruct((N_GRAPHS, N_GRAPHS), _f32),
    scratch_shapes=[
        pltpu.VMEM((N_GRAPHS, D_OUT2), _f32),
        pltpu.VMEM((N_GRAPHS, 1), _f32),
    ],
)


@jax.jit
def kernel(x, edge_index, batch, W1, V1, b1, W2, V2, b2, fc_w, fc_b):
    row = edge_index[0].astype(_i32)
    col = edge_index[1].astype(_i32)
    rowp = jnp.concatenate([row, jnp.zeros((E_PAD - E,), _i32)])
    colp = jnp.concatenate([col, jnp.full((E_PAD - E,), DUMMY, _i32)])
    rowp = rowp.reshape(E_PAD // CHUNK, CHUNK)
    colp = colp.reshape(E_PAD // CHUNK, CHUNK)

    ones8 = jnp.ones((CHUNK, DEG_W), _f32)
    zeros8 = jnp.zeros((CHUNK, DEG_W), _f32)
    zeros256b = jnp.zeros((CHUNK, D_HID), _bf16)
    zeros64 = jnp.zeros((CHUNK, D_OUT2), _bf16)

    deg2 = _deg_pass(colp, ones8, zeros8)
    dega = deg2[0, :N, 0:1]
    degb = deg2[1, :N, 0:1]

    xv1 = _mmv(x, V1)
    h1 = _mm1(x, W1, dega, degb)
    agg1 = _agg1_pass(h1, h1, rowp, colp, zeros256b)
    h2, xv2 = _mm2(agg1[0], agg1[1], xv1, dega, degb,
                   b1.reshape(1, -1), W2, V2)
    agg2 = _agg2_pass(h2, h2, rowp, colp, zeros64)

    batch3d = batch.astype(_i32).reshape(GRID_M, 1, TILE_M)
    fcw_pad = jnp.zeros((D_OUT2, N_GRAPHS), _f32).at[:, :fc_w.shape[1]].set(fc_w)
    fcb_pad = jnp.zeros((1, N_GRAPHS), _f32).at[0, :fc_b.shape[0]].set(fc_b)
    out_pad = _final(agg2[0], agg2[1], xv2, dega, degb, b2.reshape(1, -1),
                     batch3d, fcw_pad, fcb_pad)
    return out_pad[:, :fc_b.shape[0]]
